# baseline (speedup 1.0000x reference)
import functools

import jax
import jax.numpy as jnp
from jax import lax
from jax.experimental import pallas as pl
from jax.experimental.pallas import tpu as pltpu
from jax.experimental.pallas import tpu_sc as plsc

_NW = 32


@functools.lru_cache(maxsize=1)
def _sc_mesh():
    return plsc.VectorSubcoreMesh(core_axis_name="c", subcore_axis_name="s")

N = 10000
E = 320000
H = 64
DEPTH = 3

N32 = 10240


def _feat(ins, W, pre, post):
    n = ins[0].shape[0]
    Fout = W.shape[1]
    nin = len(ins)
    has_pre = pre is not None
    has_post = post is not None

    def body(*refs):
        o_ref = refs[-1]
        z = refs[0][...]
        for t in range(1, nin):
            z = z + refs[t][...]
        i = nin
        w = refs[i][...]; i += 1
        if has_pre:
            z = z * refs[i][...]; i += 1
        xw = jnp.dot(z, w, preferred_element_type=jnp.float32)
        if has_post:
            xw = xw * refs[i][...]; i += 1
        o_ref[...] = xw

    args = list(ins) + [W] + ([pre] if has_pre else []) + ([post] if has_post else [])
    return pl.pallas_call(
        body, out_shape=jax.ShapeDtypeStruct((n, Fout), jnp.float32))(*args)


def _score(h, w2d, n_real):
    n = h.shape[0]

    def body(h_ref, w_ref, o_ref):
        w = w_ref[...]
        inv = lax.rsqrt(jnp.sum(w * w))
        s = jnp.tanh(jnp.dot(h_ref[...], w, preferred_element_type=jnp.float32) * inv)
        rows = lax.broadcasted_iota(jnp.int32, (n, 1), 0)
        o_ref[...] = jnp.where(rows < n_real, s, -2.0)

    return pl.pallas_call(
        body, out_shape=jax.ShapeDtypeStruct((n, 1), jnp.float32))(h, w2d)


def _rank(s2d, k):
    R = s2d.shape[0]

    def body(s_ref, o_ref):
        s = s_ref[...]
        a = s[:, :, None]
        fi = (lax.broadcasted_iota(jnp.int32, (R, 128), 0) * 128
              + lax.broadcasted_iota(jnp.int32, (R, 128), 1))[:, :, None]
        o_ref[...] = jnp.zeros((R, 128), jnp.int32)

        def step(j, _):
            row = s_ref[pl.ds(j, 1), :].reshape(1, 1, 128)
            fj = (j * 128 + lax.broadcasted_iota(jnp.int32, (1, 1, 128), 2))
            gt = jnp.sum((row > a).astype(jnp.int32), axis=2)
            eq = jnp.sum(((row == a) & (fj < fi)).astype(jnp.int32), axis=2)
            o_ref[...] = o_ref[...] + gt + eq
            return 0

        lax.fori_loop(0, R, step, 0)
        o_ref[...] = jnp.minimum(o_ref[...], k)

    return pl.pallas_call(
        body, out_shape=jax.ShapeDtypeStruct((R, 128), jnp.int32))(s2d)


def _bmof(kp):
    return 512 if kp % 512 == 0 else 256


def _mmbig(Rm, Ct, k, exact_bf16=False):
    kp, n = Rm.shape
    bm = _bmof(kp)
    bk = min(2048, n)
    gi, gk = kp // bm, n // bk

    def body(l_ref, r_ref, b_ref, bt_ref, acc_ref):
        i, j, kk = pl.program_id(0), pl.program_id(1), pl.program_id(2)

        @pl.when(kk == 0)
        def _():
            acc_ref[...] = jnp.zeros_like(acc_ref)

        rl = i * bm + lax.broadcasted_iota(jnp.int32, (bm, 1), 0)
        lhs = jnp.where(rl < k, l_ref[...], 0.0)
        rr = j * bm + lax.broadcasted_iota(jnp.int32, (bm, 1), 0)
        rhs = jnp.where(rr < k, r_ref[...], 0.0)
        if exact_bf16:
            lhs = lhs.astype(jnp.bfloat16)
            rhs = rhs.astype(jnp.bfloat16)
        acc_ref[...] += lax.dot_general(
            lhs, rhs, (((1,), (1,)), ((), ())), preferred_element_type=jnp.float32)

        @pl.when(kk == gk - 1)
        def _():
            a = acc_ref[...]
            rg = i * bm + lax.broadcasted_iota(jnp.int32, (bm, bm), 0)
            cg = j * bm + lax.broadcasted_iota(jnp.int32, (bm, bm), 1)
            fixed = jnp.where(rg == cg, 1.0, a)
            b_ref[...] = fixed
            bt_ref[...] = fixed.T

    return pl.pallas_call(
        body,
        grid=(gi, gi, gk),
        in_specs=[
            pl.BlockSpec((bm, bk), lambda i, j, kk: (i, kk)),
            pl.BlockSpec((bm, bk), lambda i, j, kk: (j, kk)),
        ],
        out_specs=[
            pl.BlockSpec((bm, bm), lambda i, j, kk: (i, j)),
            pl.BlockSpec((bm, bm), lambda i, j, kk: (j, i)),
        ],
        out_shape=[jax.ShapeDtypeStruct((kp, kp), jnp.float32),
                   jax.ShapeDtypeStruct((kp, kp), jnp.float32)],
        scratch_shapes=[pltpu.VMEM((bm, bm), jnp.float32)],
        compiler_params=pltpu.CompilerParams(
            dimension_semantics=("parallel", "parallel", "arbitrary")),
    )(Rm, Ct)


def _rowsum_dinv(Bt):
    kp = Bt.shape[0]
    bm = _bmof(kp)
    def body(bt_ref, o_ref):
        o_ref[...] = lax.rsqrt(jnp.sum(bt_ref[...], axis=1, keepdims=True))
    return pl.pallas_call(
        body,
        grid=(kp // bm,),
        in_specs=[pl.BlockSpec((bm, kp), lambda i: (i, 0))],
        out_specs=pl.BlockSpec((bm, 1), lambda i: (i, 0)),
        out_shape=jax.ShapeDtypeStruct((kp, 1), jnp.float32),
    )(Bt)


def _dense_combine(Bt, z, dinv, b, relu):
    kp = Bt.shape[0]
    f = z.shape[1]
    bm = _bmof(kp)
    def body(bt_ref, z_ref, d_ref, b_ref, o_ref):
        y = jnp.dot(bt_ref[...], z_ref[...], preferred_element_type=jnp.float32)
        out = d_ref[...] * y + b_ref[...]
        o_ref[...] = jnp.maximum(out, 0.0) if relu else out
    return pl.pallas_call(
        body,
        grid=(kp // bm,),
        in_specs=[
            pl.BlockSpec((bm, kp), lambda i: (i, 0)),
            pl.BlockSpec((kp, f), lambda i: (0, 0)),
            pl.BlockSpec((bm, 1), lambda i: (i, 0)),
            pl.BlockSpec((1, f), lambda i: (0, 0)),
        ],
        out_specs=pl.BlockSpec((bm, f), lambda i: (i, 0)),
        out_shape=jax.ShapeDtypeStruct((kp, f), jnp.float32),
    )(Bt, z, dinv, b)


def _sparse_combine(xs, p0, p1, dinv, b, relu):
    n, f = xs.shape
    def body(x_ref, p0_ref, p1_ref, d_ref, b_ref, o_ref):
        out = d_ref[...] * (p0_ref[...] + p1_ref[...] + x_ref[...]) + b_ref[...]
        o_ref[...] = jnp.maximum(out, 0.0) if relu else out
    return pl.pallas_call(
        body, out_shape=jax.ShapeDtypeStruct((n, f), jnp.float32))(
            xs, p0, p1, dinv, b)


def _reduce32(parts, mode):
    m = parts.shape[1]
    def body(p_ref, o_ref):
        s = jnp.sum(p_ref[...], axis=0, keepdims=True)
        if mode == "dinv":
            o_ref[...] = lax.rsqrt(s + 1.0)
        else:
            o_ref[...] = s
    return pl.pallas_call(
        body, out_shape=jax.ShapeDtypeStruct((1, m), parts.dtype if mode != "dinv" else jnp.float32))(parts)


def _iota16():
    return lax.broadcasted_iota(jnp.int32, (16,), 0)


def _sc_deg_parts(dst32):
    epw = dst32.shape[1]

    @functools.partial(
        pl.kernel, mesh=_sc_mesh(),
        compiler_params=pltpu.CompilerParams(needs_layout_passes=False, use_tc_tiling_on_sc=False),
        out_type=jax.ShapeDtypeStruct((_NW, N32), jnp.float32),
        scratch_types=[
            pltpu.VMEM((epw,), jnp.int32),
            pltpu.VMEM((N32,), jnp.float32),
        ])
    def k(dst_hbm, z_hbm, out_hbm, idx_v, hist_v):
        wid = lax.axis_index("s") * 2 + lax.axis_index("c")
        pltpu.sync_copy(z_hbm, hist_v)
        pltpu.sync_copy(dst_hbm.at[wid], idx_v)
        ones = jnp.ones((16,), jnp.float32)

        def body(i, _):
            d16 = idx_v[pl.ds(i * 16, 16)]
            plsc.addupdate_scatter(hist_v, [d16], ones)
            return 0
        lax.fori_loop(0, epw // 16, body, 0)

        pltpu.sync_copy(hist_v, out_hbm.at[wid])

    return k(dst32, jnp.zeros((N32,), jnp.float32))


def _sc_scatter_conv(xs, src_r, dst_r):
    nb, bw = src_r.shape[1], src_r.shape[2]
    zrows = N32 // 16

    @functools.partial(
        pl.kernel, mesh=_sc_mesh(),
        compiler_params=pltpu.CompilerParams(needs_layout_passes=False, use_tc_tiling_on_sc=False),
        out_type=jax.ShapeDtypeStruct((2, N32, H), jnp.float32),
        scratch_types=[
            pltpu.VMEM((nb, bw), jnp.int32),
            pltpu.VMEM((nb, bw), jnp.int32),
            pltpu.VMEM((bw, H), jnp.float32),
            pltpu.VMEM_SHARED((N32, H), jnp.float32),
            pltpu.SemaphoreType.DMA,
        ])
    def k(xs_hbm, src_hbm, dst_hbm, z_hbm, out_hbm, src_v, dst_v, rows_v, acc_sh, sem):
        cid = lax.axis_index("c")
        sid = lax.axis_index("s")
        wid = sid * 2 + cid
        pltpu.sync_copy(z_hbm.at[pl.ds(sid * zrows, zrows)],
                        acc_sh.at[pl.ds(sid * zrows, zrows)])
        plsc.subcore_barrier()

        pltpu.sync_copy(src_hbm.at[wid], src_v)
        pltpu.sync_copy(dst_hbm.at[wid], dst_v)

        def body(j, _):
            pltpu.async_copy(xs_hbm.at[src_v.at[j]], rows_v, sem).wait()
            pltpu.sync_copy(rows_v, acc_sh.at[dst_v.at[j]], add=True)
            return 0
        lax.fori_loop(0, nb, body, 0)

        plsc.subcore_barrier()
        pltpu.sync_copy(acc_sh.at[pl.ds(sid * zrows, zrows)],
                        out_hbm.at[cid, pl.ds(sid * zrows, zrows)])

    out = k(xs, src_r, dst_r, jnp.zeros((N32, H), jnp.float32))
    return out[0], out[1]


def _sc_topk(rankc, scores, h, k, kp):
    n32 = rankc.shape[0]
    chunk = n32 // _NW
    nb = chunk // 16
    stripe = kp // 16

    @functools.partial(
        pl.kernel, mesh=_sc_mesh(),
        compiler_params=pltpu.CompilerParams(needs_layout_passes=False, use_tc_tiling_on_sc=False),
        out_type=[jax.ShapeDtypeStruct((_NW, kp), jnp.int32),
                  jax.ShapeDtypeStruct((_NW, kp), jnp.float32),
                  jax.ShapeDtypeStruct((2, kp, H), jnp.float32)],
        scratch_types=[
            pltpu.VMEM((chunk,), jnp.int32),
            pltpu.VMEM((chunk,), jnp.float32),
            pltpu.VMEM((chunk, H), jnp.float32),
            pltpu.VMEM((kp,), jnp.int32),
            pltpu.VMEM((kp,), jnp.float32),
            pltpu.VMEM((nb, 16), jnp.int32),
            pltpu.VMEM_SHARED((kp, H), jnp.float32),
            pltpu.SemaphoreType.DMA,
        ])
    def kn(rank_hbm, score_hbm, h_hbm, zi_hbm, zf_hbm, zh_hbm,
           perm_o, vals_o, hg_o,
           rank_v, score_v, h_v, perm_v, vals_v, idx_buf, hg_sh, sem):
        cid = lax.axis_index("c")
        sid = lax.axis_index("s")
        wid = sid * 2 + cid
        wbase = wid * chunk
        pltpu.sync_copy(zi_hbm, perm_v)
        pltpu.sync_copy(zf_hbm, vals_v)
        pltpu.sync_copy(zh_hbm.at[pl.ds(sid * stripe, stripe)],
                        hg_sh.at[pl.ds(sid * stripe, stripe)])
        plsc.subcore_barrier()
        pltpu.sync_copy(rank_hbm.at[pl.ds(wbase, chunk)], rank_v)
        pltpu.sync_copy(score_hbm.at[pl.ds(wbase, chunk)], score_v)
        pltpu.sync_copy(h_hbm.at[pl.ds(wbase, chunk)], h_v)
        for b in range(nb):
            r16 = rank_v[pl.ds(b * 16, 16)]
            s16 = score_v[pl.ds(b * 16, 16)]
            i16 = wbase + b * 16 + _iota16()
            m = r16 < k
            plsc.addupdate_scatter(perm_v, [r16], i16, mask=m)
            plsc.addupdate_scatter(vals_v, [r16], s16, mask=m)
            idx_buf[b, :] = jnp.minimum(r16, k)
            pltpu.sync_copy(h_v.at[pl.ds(b * 16, 16)],
                            hg_sh.at[idx_buf.at[b]], add=True)
        plsc.subcore_barrier()
        pltpu.sync_copy(perm_v, perm_o.at[wid])
        pltpu.sync_copy(vals_v, vals_o.at[wid])
        pltpu.sync_copy(hg_sh.at[pl.ds(sid * stripe, stripe)],
                        hg_o.at[cid, pl.ds(sid * stripe, stripe)])

    return kn(rankc, scores, h,
              jnp.zeros((kp,), jnp.int32), jnp.zeros((kp,), jnp.float32),
              jnp.zeros((kp, H), jnp.float32))


def _sc_row_gather2(T0, T1, perm):
    kp = perm.shape[0]
    m = T0.shape[1]
    chunk = kp // _NW
    nbb = chunk // 8

    @functools.partial(
        pl.kernel, mesh=_sc_mesh(),
        compiler_params=pltpu.CompilerParams(needs_layout_passes=False, use_tc_tiling_on_sc=False),
        out_type=[jax.ShapeDtypeStruct((kp, m), jnp.float32),
                  jax.ShapeDtypeStruct((kp, m), jnp.float32)],
        scratch_types=[
            pltpu.VMEM((chunk,), jnp.int32),
            pltpu.VMEM((8, m), jnp.float32),
            pltpu.VMEM((8, m), jnp.float32),
            pltpu.SemaphoreType.DMA,
            pltpu.SemaphoreType.DMA,
        ])
    def kn(t0_hbm, t1_hbm, perm_hbm, o0_hbm, o1_hbm, idx_v, b0_v, b1_v, s0, s1):
        wid = lax.axis_index("s") * 2 + lax.axis_index("c")
        wbase = wid * chunk
        pltpu.sync_copy(perm_hbm.at[pl.ds(wbase, chunk)], idx_v)

        def body(b, _):
            c0 = pltpu.async_copy(t0_hbm.at[idx_v.at[pl.ds(b * 8, 8)]], b0_v, s0)
            c1 = pltpu.async_copy(t1_hbm.at[idx_v.at[pl.ds(b * 8, 8)]], b1_v, s1)
            c0.wait()
            pltpu.sync_copy(b0_v, o0_hbm.at[pl.ds(wbase + b * 8, 8)])
            c1.wait()
            pltpu.sync_copy(b1_v, o1_hbm.at[pl.ds(wbase + b * 8, 8)])
            return 0
        lax.fori_loop(0, nbb, body, 0)

    return kn(T0, T1, perm)


def _sc_edge_ranks(src32, dst32, rankc):
    epw = src32.shape[1]

    @functools.partial(
        pl.kernel, mesh=_sc_mesh(),
        compiler_params=pltpu.CompilerParams(needs_layout_passes=False, use_tc_tiling_on_sc=False),
        out_type=[jax.ShapeDtypeStruct((_NW, epw), jnp.int32),
                  jax.ShapeDtypeStruct((_NW, epw), jnp.int32)],
        scratch_types=[
            pltpu.VMEM((N32,), jnp.int32),
            pltpu.VMEM((epw,), jnp.int32),
            pltpu.VMEM((epw,), jnp.int32),
            pltpu.VMEM((epw,), jnp.int32),
            pltpu.VMEM((epw,), jnp.int32),
        ])
    def kn(src_hbm, dst_hbm, rank_hbm, rs_o, rd_o, rk_v, s_v, d_v, rs_v, rd_v):
        wid = lax.axis_index("s") * 2 + lax.axis_index("c")
        pltpu.sync_copy(rank_hbm, rk_v)
        pltpu.sync_copy(src_hbm.at[wid], s_v)
        pltpu.sync_copy(dst_hbm.at[wid], d_v)

        def body(i, _):
            sl = pl.ds(i * 16, 16)
            s16 = s_v[sl]
            d16 = d_v[sl]
            rs_v[sl] = plsc.load_gather(rk_v, [s16]) * 16384 + d16
            rd_v[sl] = plsc.load_gather(rk_v, [d16]) * 16384 + s16
            return 0
        lax.fori_loop(0, epw // 16, body, 0)
        pltpu.sync_copy(rs_v, rs_o.at[wid])
        pltpu.sync_copy(rd_v, rd_o.at[wid])

    return kn(src32, dst32, rankc)


def _sc_bucket(keys32):
    epw = keys32.shape[1]
    REG = 12288
    DUMMY = 0x7F000000

    @functools.partial(
        pl.kernel, mesh=_sc_mesh(),
        compiler_params=pltpu.CompilerParams(needs_layout_passes=False, use_tc_tiling_on_sc=False),
        out_type=[jax.ShapeDtypeStruct((_NW, REG), jnp.int32),
                  jax.ShapeDtypeStruct((_NW, 32), jnp.int32)],
        scratch_types=[
            pltpu.VMEM((epw,), jnp.int32),
            pltpu.VMEM((REG,), jnp.int32),
            pltpu.VMEM((32,), jnp.int32),
        ])
    def kn(key_hbm, sort_o, off_o, keys_v, out_v, off_v):
        wid = lax.axis_index("s") * 2 + lax.axis_index("c")
        pltpu.sync_copy(key_hbm.at[wid], keys_v)
        ones_i = jnp.ones((16,), jnp.int32)
        io16 = _iota16()

        def zh(i, _):
            off_v[pl.ds(i * 16, 16)] = jnp.zeros((16,), jnp.int32)
            return 0
        lax.fori_loop(0, 2, zh, 0)

        def hist_body(i, _):
            g16 = lax.shift_right_logical(keys_v[pl.ds(i * 16, 16)], 22)
            plsc.addupdate_scatter(off_v, [g16], ones_i)
            return 0
        lax.fori_loop(0, epw // 16, hist_body, 0)

        h0 = off_v[pl.ds(0, 16)]
        h1 = off_v[pl.ds(16, 16)]
        a0 = jnp.bitwise_and(h0 + 7, -8)
        a1 = jnp.bitwise_and(h1 + 7, -8)
        i0 = plsc.cumsum(a0)
        i1 = plsc.cumsum(a1)
        tot0 = jnp.max(i0)
        e0 = i0 - a0
        e1 = i1 - a1 + tot0
        off_v[pl.ds(0, 16)] = e0
        off_v[pl.ds(16, 16)] = e1

        dummy16 = jnp.full((16,), DUMMY, jnp.int32)

        def pf(i, _):
            out_v[pl.ds(i * 16, 16)] = dummy16
            return 0
        lax.fori_loop(0, REG // 16, pf, 0)

        for g in range(20):
            evec = e0 if g < 16 else e1
            start = jnp.sum(jnp.where(io16 == (g % 16), evec, 0))
            c0 = jnp.zeros((16,), jnp.int32) + start

            def app(i, c, g=g):
                k16 = keys_v[pl.ds(i * 16, 16)]
                m = lax.shift_right_logical(k16, 22) == g
                incl = plsc.cumsum(jnp.where(m, ones_i, 0))
                pos16 = c + incl - 1
                plsc.store_scatter(out_v, [jnp.where(m, pos16, REG - 1)],
                                   k16, mask=m)
                return c + plsc.all_reduce_population_count(m)
            c0 = lax.fori_loop(0, epw // 16, app, c0)

        pltpu.sync_copy(out_v, sort_o.at[wid])
        pltpu.sync_copy(off_v, off_o.at[wid])

    return kn(keys32)


def _sc_build_RC_sorted(sorted_keys, offs, perm, k, kp):
    RPT = 8
    npass = kp // (_NW * RPT)
    REG = sorted_keys.shape[1]
    CH = 2048
    slabw = RPT * N32

    @functools.partial(
        pl.kernel, mesh=_sc_mesh(),
        compiler_params=pltpu.CompilerParams(needs_layout_passes=False, use_tc_tiling_on_sc=False),
        out_type=jax.ShapeDtypeStruct((kp * N32,), jnp.float32),
        scratch_types=[
            pltpu.VMEM((RPT * N32,), jnp.float32),
            pltpu.VMEM((_NW, 32), jnp.int32),
            pltpu.VMEM((2, CH), jnp.int32),
            pltpu.VMEM((kp,), jnp.int32),
            pltpu.SemaphoreType.DMA,
        ])
    def kn(sort_hbm, off_hbm, perm_hbm, out_hbm, slab_v, offs_v, ch_v, perm_v, sem):
        wid = lax.axis_index("s") * 2 + lax.axis_index("c")
        pltpu.sync_copy(off_hbm, offs_v)
        pltpu.sync_copy(perm_hbm, perm_v)
        ones = jnp.ones((16,), jnp.float32)
        z16 = jnp.zeros((16,), jnp.float32)
        io16 = _iota16()

        def do_pass(p, _):
            slab_lo = (p * _NW + wid) * RPT
            flat_lo = slab_lo * N32

            def zr(i, _2):
                slab_v[pl.ds(i * 16, 16)] = z16
                return 0
            lax.fori_loop(0, slabw // 16, zr, 0)

            for w in range(_NW):
                lo16 = offs_v[w, pl.ds(0, 16)]
                hi16 = offs_v[w, pl.ds(16, 16)]
                start = (jnp.sum(jnp.where(io16 == p, lo16, 0))
                         + jnp.sum(jnp.where(io16 + 16 == p, hi16, 0)))
                pn = p + 1
                end = (jnp.sum(jnp.where(io16 == pn, lo16, 0))
                       + jnp.sum(jnp.where(io16 + 16 == pn, hi16, 0)))
                nch = (end - start + CH - 1) // CH

                def fetch(c, buf, w=w, start=start):
                    ofs = pl.multiple_of(start + c * CH, 8)
                    pltpu.async_copy(sort_hbm.at[w, pl.ds(ofs, CH)],
                                     ch_v.at[buf], sem)

                @pl.when(nch > 0)
                def _(w=w, start=start, nch=nch):
                    fetch(0, 0)

                    def ch_body(c, _2):
                        buf = lax.rem(c, 2)
                        pltpu.make_async_copy(sort_hbm.at[w, pl.ds(0, CH)],
                                              ch_v.at[buf], sem).wait()

                        @pl.when(c + 1 < nch)
                        def _():
                            fetch(c + 1, 1 - buf)

                        def b_body(i, _3):
                            k16 = ch_v[buf, pl.ds(i * 16, 16)]
                            rk = lax.shift_right_logical(k16, 14)
                            col = jnp.bitwise_and(k16, 16383)
                            mm = (rk >= slab_lo) & (rk < slab_lo + RPT)
                            loc = (rk - slab_lo) * N32 + col
                            plsc.addupdate_scatter(
                                slab_v, [jnp.where(mm, loc, 0)], ones, mask=mm)
                            return 0
                        lax.fori_loop(0, CH // 16, b_body, 0)
                        return 0
                    lax.fori_loop(0, nch, ch_body, 0)

            def sl_body(si, _2):
                r16 = si * 16 + _iota16()
                p16 = perm_v[pl.ds(si * 16, 16)]
                loc = (r16 - slab_lo) * N32 + p16
                mm = (r16 >= slab_lo) & (r16 < slab_lo + RPT) & (r16 < k)
                plsc.addupdate_scatter(
                    slab_v, [jnp.where(mm, loc, 0)], ones, mask=mm)
                return 0
            lax.fori_loop(0, kp // 16, sl_body, 0)

            pltpu.sync_copy(slab_v, out_hbm.at[pl.ds(flat_lo, slabw)])
            return 0
        lax.fori_loop(0, npass, do_pass, 0)

    return kn(sorted_keys, offs, perm)


def _sc_up_scatter(pos, h, k, n32, dummy):
    kp = pos.shape[0]
    chunk = kp // _NW
    nb = (chunk + 15) // 16
    pad = nb * 16
    stripe = n32 // 16

    @functools.partial(
        pl.kernel, mesh=_sc_mesh(),
        compiler_params=pltpu.CompilerParams(needs_layout_passes=False, use_tc_tiling_on_sc=False),
        out_type=jax.ShapeDtypeStruct((2, n32, H), jnp.float32),
        scratch_types=[
            pltpu.VMEM((pad,), jnp.int32),
            pltpu.VMEM((pad, H), jnp.float32),
            pltpu.VMEM((nb, 16), jnp.int32),
            pltpu.VMEM_SHARED((n32, H), jnp.float32),
            pltpu.SemaphoreType.DMA,
        ])
    def kn(pos_hbm, h_hbm, z_hbm, out_hbm, pos_v, h_v, idx_buf, acc_sh, sem):
        cid = lax.axis_index("c")
        sid = lax.axis_index("s")
        wid = sid * 2 + cid
        wbase = wid * chunk
        pltpu.sync_copy(z_hbm.at[pl.ds(sid * stripe, stripe)],
                        acc_sh.at[pl.ds(sid * stripe, stripe)])
        plsc.subcore_barrier()
        pltpu.sync_copy(pos_hbm.at[pl.ds(wbase, chunk)], pos_v.at[pl.ds(0, chunk)])
        pltpu.sync_copy(h_hbm.at[pl.ds(wbase, chunk)], h_v.at[pl.ds(0, chunk)])
        for b in range(nb):
            local16 = b * 16 + _iota16()
            p16 = pos_v[pl.ds(b * 16, 16)]
            sel = ((wbase + local16) < k) & (local16 < chunk)
            idx_buf[b, :] = jnp.where(sel, p16, dummy)
            pltpu.sync_copy(h_v.at[pl.ds(b * 16, 16)],
                            acc_sh.at[idx_buf.at[b]], add=True)
        plsc.subcore_barrier()
        pltpu.sync_copy(acc_sh.at[pl.ds(sid * stripe, stripe)],
                        out_hbm.at[cid, pl.ds(sid * stripe, stripe)])

    out = kn(pos, h, jnp.zeros((n32, H), jnp.float32))
    return out[0], out[1]


def _topk_level(h, pool_w, n_real, k, kp):
    n32 = h.shape[0]
    s_col = _score(h, pool_w.reshape(H, 1), n_real)
    s2d = s_col.reshape(n32 // 128, 128)
    rankc2d = _rank(s2d, k)
    rankc = rankc2d.reshape(n32)
    scores = s_col.reshape(n32)
    perm_p, vals_p, hg = _sc_topk(rankc, scores, h, k, kp)
    perm = _reduce32(perm_p, "sum").reshape(kp)
    vals = _reduce32(vals_p, "sum").reshape(kp)
    return rankc, perm, vals, hg[0], hg[1]


def kernel(x, edge_index, edge_attr, params):
    src, dst = edge_index[0], edge_index[1]
    src_r = src.reshape(32, 125, 80)
    dst_r = dst.reshape(32, 125, 80)

    deg_parts = _sc_deg_parts(dst.reshape(32, E // 32))
    dinv = _reduce32(deg_parts, "dinv").reshape(N32, 1)

    x_pad = jnp.zeros((N32, x.shape[1]), jnp.float32).at[:N, :].set(x)

    def gcn_sparse(ins, W, b, relu, pre=None):
        Wp = W
        bp = b
        if W.shape[1] < H:
            Wp = jnp.zeros((W.shape[0], H), jnp.float32).at[:, :W.shape[1]].set(W)
            bp = jnp.zeros((H,), jnp.float32).at[:W.shape[1]].set(b)
        xs = _feat(ins, Wp, pre, dinv)
        p0, p1 = _sc_scatter_conv(xs, src_r, dst_r)
        return _sparse_combine(xs, p0, p1, dinv, bp.reshape(1, H), relu)

    h = gcn_sparse([x_pad], params['init_W'], params['init_b'], relu=True)
    h = gcn_sparse([h], params['down_W'][0], params['down_b'][0], relu=True)
    res0 = h

    k1, kp1 = 5000, 5120
    rank1, perm1, vals1, hg0, hg1 = _topk_level(h, params['pool_w'][0], N, k1, kp1)
    rks, rkd = _sc_edge_ranks(src.reshape(32, E // 32), dst.reshape(32, E // 32), rank1)
    srt_s, off_s = _sc_bucket(rks)
    srt_d, off_d = _sc_bucket(rkd)
    Rm = _sc_build_RC_sorted(srt_s, off_s, perm1, k1, kp1).reshape(kp1, N32)
    Ct = _sc_build_RC_sorted(srt_d, off_d, perm1, k1, kp1).reshape(kp1, N32)
    B1, Bt1 = _mmbig(Rm, Ct, k1, exact_bf16=True)
    dinv1 = _rowsum_dinv(Bt1)
    z = _feat([hg0, hg1], params['down_W'][1], vals1.reshape(kp1, 1), dinv1)
    h = _dense_combine(Bt1, z, dinv1, params['down_b'][1].reshape(1, H), relu=True)
    res1 = h

    k2, kp2 = 2500, 2560
    rank2, perm2, vals2, hg0, hg1 = _topk_level(h, params['pool_w'][1], k1, k2, kp2)
    R2, Ct2 = _sc_row_gather2(B1, Bt1, perm2)
    B2, Bt2 = _mmbig(R2, Ct2, k2)
    dinv2 = _rowsum_dinv(Bt2)
    z = _feat([hg0, hg1], params['down_W'][2], vals2.reshape(kp2, 1), dinv2)
    h = _dense_combine(Bt2, z, dinv2, params['down_b'][2].reshape(1, H), relu=True)
    res2 = h

    k3, kp3 = 1250, 1280
    rank3, perm3, vals3, hg0, hg1 = _topk_level(h, params['pool_w'][2], k2, k3, kp3)
    R3, Ct3 = _sc_row_gather2(B2, Bt2, perm3)
    B3, Bt3 = _mmbig(R3, Ct3, k3)
    dinv3 = _rowsum_dinv(Bt3)
    z = _feat([hg0, hg1], params['down_W'][3], vals3.reshape(kp3, 1), dinv3)
    h = _dense_combine(Bt3, z, dinv3, params['down_b'][3].reshape(1, H), relu=True)

    p0, p1 = _sc_up_scatter(perm3, h, k3, kp2, k2)
    z = _feat([res2, p0, p1], params['up_W'][0], None, dinv2)
    h = _dense_combine(Bt2, z, dinv2, params['up_b'][0].reshape(1, H), relu=True)

    p0, p1 = _sc_up_scatter(perm2, h, k2, kp1, k1)
    z = _feat([res1, p0, p1], params['up_W'][1], None, dinv1)
    h = _dense_combine(Bt1, z, dinv1, params['up_b'][1].reshape(1, H), relu=True)

    p0, p1 = _sc_up_scatter(perm1, h, k1, N32, N)
    h = gcn_sparse([res0, p0, p1], params['up_W'][2], params['up_b'][2], relu=False)

    h = gcn_sparse([h], params['W1'], params['b1'], relu=True)
    h = gcn_sparse([h], params['W2'], params['b2'], relu=False)

    return h[:N, :3].reshape(1, -1)


# speedup vs baseline: 1.2465x; 1.2465x over previous
import functools

import jax
import jax.numpy as jnp
from jax import lax
from jax.experimental import pallas as pl
from jax.experimental.pallas import tpu as pltpu
from jax.experimental.pallas import tpu_sc as plsc

_NW = 32


@functools.lru_cache(maxsize=1)
def _sc_mesh():
    return plsc.VectorSubcoreMesh(core_axis_name="c", subcore_axis_name="s")

N = 10000
E = 320000
H = 64
DEPTH = 3

N32 = 10240


def _feat(ins, W, pre, post):
    n = ins[0].shape[0]
    Fout = W.shape[1]
    nin = len(ins)
    has_pre = pre is not None
    has_post = post is not None

    def body(*refs):
        o_ref = refs[-1]
        z = refs[0][...]
        for t in range(1, nin):
            z = z + refs[t][...]
        i = nin
        w = refs[i][...]; i += 1
        if has_pre:
            z = z * refs[i][...]; i += 1
        xw = jnp.dot(z, w, preferred_element_type=jnp.float32)
        if has_post:
            xw = xw * refs[i][...]; i += 1
        o_ref[...] = xw

    args = list(ins) + [W] + ([pre] if has_pre else []) + ([post] if has_post else [])
    return pl.pallas_call(
        body, out_shape=jax.ShapeDtypeStruct((n, Fout), jnp.float32))(*args)


def _score(h, w2d, n_real):
    n = h.shape[0]

    def body(h_ref, w_ref, o_ref):
        w = w_ref[...]
        inv = lax.rsqrt(jnp.sum(w * w))
        s = jnp.tanh(jnp.dot(h_ref[...], w, preferred_element_type=jnp.float32) * inv)
        rows = lax.broadcasted_iota(jnp.int32, (n, 1), 0)
        o_ref[...] = jnp.where(rows < n_real, s, -2.0)

    return pl.pallas_call(
        body, out_shape=jax.ShapeDtypeStruct((n, 1), jnp.float32))(h, w2d)


def _rank(s2d, k):
    R = s2d.shape[0]

    def body(s_ref, o_ref):
        s = s_ref[...]
        a = s[:, :, None]
        fi = (lax.broadcasted_iota(jnp.int32, (R, 128), 0) * 128
              + lax.broadcasted_iota(jnp.int32, (R, 128), 1))[:, :, None]
        o_ref[...] = jnp.zeros((R, 128), jnp.int32)

        def step(j, _):
            row = s_ref[pl.ds(j, 1), :].reshape(1, 1, 128)
            fj = (j * 128 + lax.broadcasted_iota(jnp.int32, (1, 1, 128), 2))
            gt = jnp.sum((row > a).astype(jnp.int32), axis=2)
            eq = jnp.sum(((row == a) & (fj < fi)).astype(jnp.int32), axis=2)
            o_ref[...] = o_ref[...] + gt + eq
            return 0

        lax.fori_loop(0, R, step, 0)
        o_ref[...] = jnp.minimum(o_ref[...], k)

    return pl.pallas_call(
        body, out_shape=jax.ShapeDtypeStruct((R, 128), jnp.int32))(s2d)


def _bmof(kp):
    return 512 if kp % 512 == 0 else 256


def _mmbig(Rm, Ct, k, exact_bf16=False):
    kp, n = Rm.shape
    bm = _bmof(kp)
    bk = min(2048, n)
    gi, gk = kp // bm, n // bk

    def body(l_ref, r_ref, b_ref, bt_ref, acc_ref):
        i, j, kk = pl.program_id(0), pl.program_id(1), pl.program_id(2)

        @pl.when(kk == 0)
        def _():
            acc_ref[...] = jnp.zeros_like(acc_ref)

        rl = i * bm + lax.broadcasted_iota(jnp.int32, (bm, 1), 0)
        lhs = jnp.where(rl < k, l_ref[...], 0.0)
        rr = j * bm + lax.broadcasted_iota(jnp.int32, (bm, 1), 0)
        rhs = jnp.where(rr < k, r_ref[...], 0.0)
        if exact_bf16:
            lhs = lhs.astype(jnp.bfloat16)
            rhs = rhs.astype(jnp.bfloat16)
        acc_ref[...] += lax.dot_general(
            lhs, rhs, (((1,), (1,)), ((), ())), preferred_element_type=jnp.float32)

        @pl.when(kk == gk - 1)
        def _():
            a = acc_ref[...]
            rg = i * bm + lax.broadcasted_iota(jnp.int32, (bm, bm), 0)
            cg = j * bm + lax.broadcasted_iota(jnp.int32, (bm, bm), 1)
            fixed = jnp.where(rg == cg, 1.0, a)
            b_ref[...] = fixed
            bt_ref[...] = fixed.T

    return pl.pallas_call(
        body,
        grid=(gi, gi, gk),
        in_specs=[
            pl.BlockSpec((bm, bk), lambda i, j, kk: (i, kk)),
            pl.BlockSpec((bm, bk), lambda i, j, kk: (j, kk)),
        ],
        out_specs=[
            pl.BlockSpec((bm, bm), lambda i, j, kk: (i, j)),
            pl.BlockSpec((bm, bm), lambda i, j, kk: (j, i)),
        ],
        out_shape=[jax.ShapeDtypeStruct((kp, kp), jnp.float32),
                   jax.ShapeDtypeStruct((kp, kp), jnp.float32)],
        scratch_shapes=[pltpu.VMEM((bm, bm), jnp.float32)],
        compiler_params=pltpu.CompilerParams(
            dimension_semantics=("parallel", "parallel", "arbitrary")),
    )(Rm, Ct)


def _rowsum_dinv(Bt):
    kp = Bt.shape[0]
    bm = _bmof(kp)
    def body(bt_ref, o_ref):
        o_ref[...] = lax.rsqrt(jnp.sum(bt_ref[...], axis=1, keepdims=True))
    return pl.pallas_call(
        body,
        grid=(kp // bm,),
        in_specs=[pl.BlockSpec((bm, kp), lambda i: (i, 0))],
        out_specs=pl.BlockSpec((bm, 1), lambda i: (i, 0)),
        out_shape=jax.ShapeDtypeStruct((kp, 1), jnp.float32),
    )(Bt)


def _dense_combine(Bt, z, dinv, b, relu):
    kp = Bt.shape[0]
    f = z.shape[1]
    bm = _bmof(kp)
    def body(bt_ref, z_ref, d_ref, b_ref, o_ref):
        y = jnp.dot(bt_ref[...], z_ref[...], preferred_element_type=jnp.float32)
        out = d_ref[...] * y + b_ref[...]
        o_ref[...] = jnp.maximum(out, 0.0) if relu else out
    return pl.pallas_call(
        body,
        grid=(kp // bm,),
        in_specs=[
            pl.BlockSpec((bm, kp), lambda i: (i, 0)),
            pl.BlockSpec((kp, f), lambda i: (0, 0)),
            pl.BlockSpec((bm, 1), lambda i: (i, 0)),
            pl.BlockSpec((1, f), lambda i: (0, 0)),
        ],
        out_specs=pl.BlockSpec((bm, f), lambda i: (i, 0)),
        out_shape=jax.ShapeDtypeStruct((kp, f), jnp.float32),
    )(Bt, z, dinv, b)


def _sparse_combine(xs, p0, p1, dinv, b, relu):
    n, f = xs.shape
    def body(x_ref, p0_ref, p1_ref, d_ref, b_ref, o_ref):
        out = d_ref[...] * (p0_ref[...] + p1_ref[...] + x_ref[...]) + b_ref[...]
        o_ref[...] = jnp.maximum(out, 0.0) if relu else out
    return pl.pallas_call(
        body, out_shape=jax.ShapeDtypeStruct((n, f), jnp.float32))(
            xs, p0, p1, dinv, b)


def _reduce32(parts, mode):
    m = parts.shape[1]
    def body(p_ref, o_ref):
        s = jnp.sum(p_ref[...], axis=0, keepdims=True)
        if mode == "dinv":
            o_ref[...] = lax.rsqrt(s + 1.0)
        else:
            o_ref[...] = s
    return pl.pallas_call(
        body, out_shape=jax.ShapeDtypeStruct((1, m), parts.dtype if mode != "dinv" else jnp.float32))(parts)


def _iota16():
    return lax.broadcasted_iota(jnp.int32, (16,), 0)


def _sc_deg_parts(dst32):
    epw = dst32.shape[1]

    @functools.partial(
        pl.kernel, mesh=_sc_mesh(),
        compiler_params=pltpu.CompilerParams(needs_layout_passes=False, use_tc_tiling_on_sc=False),
        out_type=jax.ShapeDtypeStruct((_NW, N32), jnp.float32),
        scratch_types=[
            pltpu.VMEM((epw,), jnp.int32),
            pltpu.VMEM((N32,), jnp.float32),
        ])
    def k(dst_hbm, z_hbm, out_hbm, idx_v, hist_v):
        wid = lax.axis_index("s") * 2 + lax.axis_index("c")
        pltpu.sync_copy(z_hbm, hist_v)
        pltpu.sync_copy(dst_hbm.at[wid], idx_v)
        ones = jnp.ones((16,), jnp.float32)

        def body(i, _):
            d16 = idx_v[pl.ds(i * 16, 16)]
            plsc.addupdate_scatter(hist_v, [d16], ones)
            return 0
        lax.fori_loop(0, epw // 16, body, 0)

        pltpu.sync_copy(hist_v, out_hbm.at[wid])

    return k(dst32, jnp.zeros((N32,), jnp.float32))


def _sc_scatter_conv(xs, src_r, dst_r):
    nb, bw = src_r.shape[1], src_r.shape[2]
    zrows = N32 // 16

    @functools.partial(
        pl.kernel, mesh=_sc_mesh(),
        compiler_params=pltpu.CompilerParams(needs_layout_passes=False, use_tc_tiling_on_sc=False),
        out_type=jax.ShapeDtypeStruct((2, N32, H), jnp.float32),
        scratch_types=[
            pltpu.VMEM((nb, bw), jnp.int32),
            pltpu.VMEM((nb, bw), jnp.int32),
            pltpu.VMEM((bw, H), jnp.float32),
            pltpu.VMEM_SHARED((N32, H), jnp.float32),
            pltpu.SemaphoreType.DMA,
        ])
    def k(xs_hbm, src_hbm, dst_hbm, z_hbm, out_hbm, src_v, dst_v, rows_v, acc_sh, sem):
        cid = lax.axis_index("c")
        sid = lax.axis_index("s")
        wid = sid * 2 + cid
        pltpu.sync_copy(z_hbm.at[pl.ds(sid * zrows, zrows)],
                        acc_sh.at[pl.ds(sid * zrows, zrows)])
        plsc.subcore_barrier()

        pltpu.sync_copy(src_hbm.at[wid], src_v)
        pltpu.sync_copy(dst_hbm.at[wid], dst_v)

        def body(j, _):
            pltpu.async_copy(xs_hbm.at[src_v.at[j]], rows_v, sem).wait()
            pltpu.sync_copy(rows_v, acc_sh.at[dst_v.at[j]], add=True)
            return 0
        lax.fori_loop(0, nb, body, 0)

        plsc.subcore_barrier()
        pltpu.sync_copy(acc_sh.at[pl.ds(sid * zrows, zrows)],
                        out_hbm.at[cid, pl.ds(sid * zrows, zrows)])

    out = k(xs, src_r, dst_r, jnp.zeros((N32, H), jnp.float32))
    return out[0], out[1]


def _sc_topk(rankc, scores, h, k, kp):
    n32 = rankc.shape[0]
    chunk = n32 // _NW
    nb = chunk // 16
    stripe = kp // 16

    @functools.partial(
        pl.kernel, mesh=_sc_mesh(),
        compiler_params=pltpu.CompilerParams(needs_layout_passes=False, use_tc_tiling_on_sc=False),
        out_type=[jax.ShapeDtypeStruct((_NW, kp), jnp.int32),
                  jax.ShapeDtypeStruct((_NW, kp), jnp.float32),
                  jax.ShapeDtypeStruct((2, kp, H), jnp.float32)],
        scratch_types=[
            pltpu.VMEM((chunk,), jnp.int32),
            pltpu.VMEM((chunk,), jnp.float32),
            pltpu.VMEM((chunk, H), jnp.float32),
            pltpu.VMEM((kp,), jnp.int32),
            pltpu.VMEM((kp,), jnp.float32),
            pltpu.VMEM((nb, 16), jnp.int32),
            pltpu.VMEM_SHARED((kp, H), jnp.float32),
            pltpu.SemaphoreType.DMA,
        ])
    def kn(rank_hbm, score_hbm, h_hbm, zi_hbm, zf_hbm, zh_hbm,
           perm_o, vals_o, hg_o,
           rank_v, score_v, h_v, perm_v, vals_v, idx_buf, hg_sh, sem):
        cid = lax.axis_index("c")
        sid = lax.axis_index("s")
        wid = sid * 2 + cid
        wbase = wid * chunk
        pltpu.sync_copy(zi_hbm, perm_v)
        pltpu.sync_copy(zf_hbm, vals_v)
        pltpu.sync_copy(zh_hbm.at[pl.ds(sid * stripe, stripe)],
                        hg_sh.at[pl.ds(sid * stripe, stripe)])
        plsc.subcore_barrier()
        pltpu.sync_copy(rank_hbm.at[pl.ds(wbase, chunk)], rank_v)
        pltpu.sync_copy(score_hbm.at[pl.ds(wbase, chunk)], score_v)
        pltpu.sync_copy(h_hbm.at[pl.ds(wbase, chunk)], h_v)
        for b in range(nb):
            r16 = rank_v[pl.ds(b * 16, 16)]
            s16 = score_v[pl.ds(b * 16, 16)]
            i16 = wbase + b * 16 + _iota16()
            m = r16 < k
            plsc.addupdate_scatter(perm_v, [r16], i16, mask=m)
            plsc.addupdate_scatter(vals_v, [r16], s16, mask=m)
            idx_buf[b, :] = jnp.minimum(r16, k)
            pltpu.sync_copy(h_v.at[pl.ds(b * 16, 16)],
                            hg_sh.at[idx_buf.at[b]], add=True)
        plsc.subcore_barrier()
        pltpu.sync_copy(perm_v, perm_o.at[wid])
        pltpu.sync_copy(vals_v, vals_o.at[wid])
        pltpu.sync_copy(hg_sh.at[pl.ds(sid * stripe, stripe)],
                        hg_o.at[cid, pl.ds(sid * stripe, stripe)])

    return kn(rankc, scores, h,
              jnp.zeros((kp,), jnp.int32), jnp.zeros((kp,), jnp.float32),
              jnp.zeros((kp, H), jnp.float32))


def _sc_row_gather2(T0, T1, perm):
    kp = perm.shape[0]
    m = T0.shape[1]
    chunk = kp // _NW
    nbb = chunk // 8

    @functools.partial(
        pl.kernel, mesh=_sc_mesh(),
        compiler_params=pltpu.CompilerParams(needs_layout_passes=False, use_tc_tiling_on_sc=False),
        out_type=[jax.ShapeDtypeStruct((kp, m), jnp.float32),
                  jax.ShapeDtypeStruct((kp, m), jnp.float32)],
        scratch_types=[
            pltpu.VMEM((chunk,), jnp.int32),
            pltpu.VMEM((8, m), jnp.float32),
            pltpu.VMEM((8, m), jnp.float32),
            pltpu.SemaphoreType.DMA,
            pltpu.SemaphoreType.DMA,
        ])
    def kn(t0_hbm, t1_hbm, perm_hbm, o0_hbm, o1_hbm, idx_v, b0_v, b1_v, s0, s1):
        wid = lax.axis_index("s") * 2 + lax.axis_index("c")
        wbase = wid * chunk
        pltpu.sync_copy(perm_hbm.at[pl.ds(wbase, chunk)], idx_v)

        def body(b, _):
            c0 = pltpu.async_copy(t0_hbm.at[idx_v.at[pl.ds(b * 8, 8)]], b0_v, s0)
            c1 = pltpu.async_copy(t1_hbm.at[idx_v.at[pl.ds(b * 8, 8)]], b1_v, s1)
            c0.wait()
            pltpu.sync_copy(b0_v, o0_hbm.at[pl.ds(wbase + b * 8, 8)])
            c1.wait()
            pltpu.sync_copy(b1_v, o1_hbm.at[pl.ds(wbase + b * 8, 8)])
            return 0
        lax.fori_loop(0, nbb, body, 0)

    return kn(T0, T1, perm)


def _sc_edge_ranks(src32, dst32, rankc):
    epw = src32.shape[1]

    @functools.partial(
        pl.kernel, mesh=_sc_mesh(),
        compiler_params=pltpu.CompilerParams(needs_layout_passes=False, use_tc_tiling_on_sc=False),
        out_type=[jax.ShapeDtypeStruct((_NW, epw), jnp.int32),
                  jax.ShapeDtypeStruct((_NW, epw), jnp.int32)],
        scratch_types=[
            pltpu.VMEM((N32,), jnp.int32),
            pltpu.VMEM((epw,), jnp.int32),
            pltpu.VMEM((epw,), jnp.int32),
            pltpu.VMEM((epw,), jnp.int32),
            pltpu.VMEM((epw,), jnp.int32),
        ])
    def kn(src_hbm, dst_hbm, rank_hbm, rs_o, rd_o, rk_v, s_v, d_v, rs_v, rd_v):
        wid = lax.axis_index("s") * 2 + lax.axis_index("c")
        pltpu.sync_copy(rank_hbm, rk_v)
        pltpu.sync_copy(src_hbm.at[wid], s_v)
        pltpu.sync_copy(dst_hbm.at[wid], d_v)

        def body(i, _):
            sl = pl.ds(i * 16, 16)
            s16 = s_v[sl]
            d16 = d_v[sl]
            rs_v[sl] = plsc.load_gather(rk_v, [s16]) * 16384 + d16
            rd_v[sl] = plsc.load_gather(rk_v, [d16]) * 16384 + s16
            return 0
        lax.fori_loop(0, epw // 16, body, 0)
        pltpu.sync_copy(rs_v, rs_o.at[wid])
        pltpu.sync_copy(rd_v, rd_o.at[wid])

    return kn(src32, dst32, rankc)


def _sc_bucket(keys32):
    epw = keys32.shape[1]
    REG = 12288
    DUMMY = 0x7F000000

    @functools.partial(
        pl.kernel, mesh=_sc_mesh(),
        compiler_params=pltpu.CompilerParams(needs_layout_passes=False, use_tc_tiling_on_sc=False),
        out_type=[jax.ShapeDtypeStruct((_NW, REG), jnp.int32),
                  jax.ShapeDtypeStruct((_NW, 32), jnp.int32)],
        scratch_types=[
            pltpu.VMEM((epw,), jnp.int32),
            pltpu.VMEM((REG,), jnp.int32),
            pltpu.VMEM((32,), jnp.int32),
        ])
    def kn(key_hbm, sort_o, off_o, keys_v, out_v, off_v):
        wid = lax.axis_index("s") * 2 + lax.axis_index("c")
        pltpu.sync_copy(key_hbm.at[wid], keys_v)
        ones_i = jnp.ones((16,), jnp.int32)
        io16 = _iota16()

        def zh(i, _):
            off_v[pl.ds(i * 16, 16)] = jnp.zeros((16,), jnp.int32)
            return 0
        lax.fori_loop(0, 2, zh, 0)

        def hist_body(i, _):
            g16 = lax.shift_right_logical(keys_v[pl.ds(i * 16, 16)], 22)
            plsc.addupdate_scatter(off_v, [g16], ones_i)
            return 0
        lax.fori_loop(0, epw // 16, hist_body, 0)

        h0 = off_v[pl.ds(0, 16)]
        h1 = off_v[pl.ds(16, 16)]
        a0 = jnp.bitwise_and(h0 + 7, -8)
        a1 = jnp.bitwise_and(h1 + 7, -8)
        i0 = plsc.cumsum(a0)
        i1 = plsc.cumsum(a1)
        tot0 = jnp.max(i0)
        e0 = i0 - a0
        e1 = i1 - a1 + tot0
        off_v[pl.ds(0, 16)] = e0
        off_v[pl.ds(16, 16)] = e1

        dummy16 = jnp.full((16,), DUMMY, jnp.int32)

        def pf(i, _):
            out_v[pl.ds(i * 16, 16)] = dummy16
            return 0
        lax.fori_loop(0, REG // 16, pf, 0)

        for g in range(20):
            evec = e0 if g < 16 else e1
            start = jnp.sum(jnp.where(io16 == (g % 16), evec, 0))
            c0 = jnp.zeros((16,), jnp.int32) + start

            def app(i, c, g=g):
                k16 = keys_v[pl.ds(i * 16, 16)]
                m = lax.shift_right_logical(k16, 22) == g
                incl = plsc.cumsum(jnp.where(m, ones_i, 0))
                pos16 = c + incl - 1
                plsc.store_scatter(out_v, [jnp.where(m, pos16, REG - 1)],
                                   k16, mask=m)
                return c + plsc.all_reduce_population_count(m)
            c0 = lax.fori_loop(0, epw // 16, app, c0)

        pltpu.sync_copy(out_v, sort_o.at[wid])
        pltpu.sync_copy(off_v, off_o.at[wid])

    return kn(keys32)


def _sc_build_RC_sorted(sorted_keys, offs, perm, k, kp):
    RPT = 8
    npass = kp // (_NW * RPT)
    REG = sorted_keys.shape[1]
    CH = 1024
    slabw = RPT * N32

    @functools.partial(
        pl.kernel, mesh=_sc_mesh(),
        compiler_params=pltpu.CompilerParams(needs_layout_passes=False, use_tc_tiling_on_sc=False),
        out_type=jax.ShapeDtypeStruct((kp * N32,), jnp.float32),
        scratch_types=[
            pltpu.VMEM((RPT * N32,), jnp.float32),
            pltpu.VMEM((_NW, 32), jnp.int32),
            pltpu.VMEM((2, CH), jnp.int32),
            pltpu.VMEM((kp,), jnp.int32),
            pltpu.SemaphoreType.DMA,
        ])
    def kn(sort_hbm, off_hbm, perm_hbm, out_hbm, slab_v, offs_v, ch_v, perm_v, sem):
        wid = lax.axis_index("s") * 2 + lax.axis_index("c")
        pltpu.sync_copy(off_hbm, offs_v)
        pltpu.sync_copy(perm_hbm, perm_v)
        ones = jnp.ones((16,), jnp.float32)
        z16 = jnp.zeros((16,), jnp.float32)
        io16 = _iota16()

        def do_pass(p, _):
            slab_lo = (p * _NW + wid) * RPT
            flat_lo = slab_lo * N32

            def zr(i, _2):
                slab_v[pl.ds(i * 16, 16)] = z16
                return 0
            lax.fori_loop(0, slabw // 16, zr, 0)

            def bounds(w):
                lo16 = offs_v[w, pl.ds(0, 16)]
                hi16 = offs_v[w, pl.ds(16, 16)]
                start = (jnp.sum(jnp.where(io16 == p, lo16, 0))
                         + jnp.sum(jnp.where(io16 + 16 == p, hi16, 0)))
                pn = p + 1
                end = (jnp.sum(jnp.where(io16 == pn, lo16, 0))
                       + jnp.sum(jnp.where(io16 + 16 == pn, hi16, 0)))
                return start, end

            def fetch(w, start, c, buf):
                ofs = pl.multiple_of(start + c * CH, 8)
                pltpu.async_copy(sort_hbm.at[w, pl.ds(ofs, CH)],
                                 ch_v.at[buf], sem)

            def scan(buf, nw16):
                def b_body(i, _3):
                    k16 = ch_v[buf, pl.ds(i * 16, 16)]
                    rk = lax.shift_right_logical(k16, 14)
                    col = jnp.bitwise_and(k16, 16383)
                    mm = (rk >= slab_lo) & (rk < slab_lo + RPT)
                    loc = (rk - slab_lo) * N32 + col
                    plsc.addupdate_scatter(
                        slab_v, [jnp.where(mm, loc, 0)], ones, mask=mm)
                    return 0
                lax.fori_loop(0, nw16, b_body, 0)

            s0, e0b = bounds(0)
            fetch(0, s0, 0, 0)
            start, end = s0, e0b
            for w in range(_NW):
                buf = w % 2
                pltpu.make_async_copy(sort_hbm.at[w, pl.ds(0, CH)],
                                      ch_v.at[buf], sem).wait()
                if w + 1 < _NW:
                    nstart, nend = bounds(w + 1)
                    fetch(w + 1, nstart, 0, 1 - buf)
                nw = end - start
                scan(buf, (jnp.minimum(nw, CH) + 15) // 16)

                nch = (nw + CH - 1) // CH

                def extra(c, _2, w=w, start=start, buf=buf, nw=nw):
                    ofs = pl.multiple_of(start + (c + 1) * CH, 8)
                    pltpu.sync_copy(sort_hbm.at[w, pl.ds(ofs, CH)], ch_v.at[buf])
                    scan(buf, (jnp.minimum(nw - (c + 1) * CH, CH) + 15) // 16)
                    return 0
                lax.fori_loop(0, jnp.maximum(nch - 1, 0), extra, 0)
                if w + 1 < _NW:
                    start, end = nstart, nend

            def sl_body(si, _2):
                r16 = si * 16 + _iota16()
                p16 = perm_v[pl.ds(si * 16, 16)]
                loc = (r16 - slab_lo) * N32 + p16
                mm = (r16 >= slab_lo) & (r16 < slab_lo + RPT) & (r16 < k)
                plsc.addupdate_scatter(
                    slab_v, [jnp.where(mm, loc, 0)], ones, mask=mm)
                return 0
            lax.fori_loop(0, kp // 16, sl_body, 0)

            pltpu.sync_copy(slab_v, out_hbm.at[pl.ds(flat_lo, slabw)])
            return 0
        lax.fori_loop(0, npass, do_pass, 0)

    return kn(sorted_keys, offs, perm)


def _sc_up_scatter(pos, h, k, n32, dummy):
    kp = pos.shape[0]
    chunk = kp // _NW
    nb = (chunk + 15) // 16
    pad = nb * 16
    stripe = n32 // 16

    @functools.partial(
        pl.kernel, mesh=_sc_mesh(),
        compiler_params=pltpu.CompilerParams(needs_layout_passes=False, use_tc_tiling_on_sc=False),
        out_type=jax.ShapeDtypeStruct((2, n32, H), jnp.float32),
        scratch_types=[
            pltpu.VMEM((pad,), jnp.int32),
            pltpu.VMEM((pad, H), jnp.float32),
            pltpu.VMEM((nb, 16), jnp.int32),
            pltpu.VMEM_SHARED((n32, H), jnp.float32),
            pltpu.SemaphoreType.DMA,
        ])
    def kn(pos_hbm, h_hbm, z_hbm, out_hbm, pos_v, h_v, idx_buf, acc_sh, sem):
        cid = lax.axis_index("c")
        sid = lax.axis_index("s")
        wid = sid * 2 + cid
        wbase = wid * chunk
        pltpu.sync_copy(z_hbm.at[pl.ds(sid * stripe, stripe)],
                        acc_sh.at[pl.ds(sid * stripe, stripe)])
        plsc.subcore_barrier()
        pltpu.sync_copy(pos_hbm.at[pl.ds(wbase, chunk)], pos_v.at[pl.ds(0, chunk)])
        pltpu.sync_copy(h_hbm.at[pl.ds(wbase, chunk)], h_v.at[pl.ds(0, chunk)])
        for b in range(nb):
            local16 = b * 16 + _iota16()
            p16 = pos_v[pl.ds(b * 16, 16)]
            sel = ((wbase + local16) < k) & (local16 < chunk)
            idx_buf[b, :] = jnp.where(sel, p16, dummy)
            pltpu.sync_copy(h_v.at[pl.ds(b * 16, 16)],
                            acc_sh.at[idx_buf.at[b]], add=True)
        plsc.subcore_barrier()
        pltpu.sync_copy(acc_sh.at[pl.ds(sid * stripe, stripe)],
                        out_hbm.at[cid, pl.ds(sid * stripe, stripe)])

    out = kn(pos, h, jnp.zeros((n32, H), jnp.float32))
    return out[0], out[1]


def _topk_level(h, pool_w, n_real, k, kp):
    n32 = h.shape[0]
    s_col = _score(h, pool_w.reshape(H, 1), n_real)
    s2d = s_col.reshape(n32 // 128, 128)
    rankc2d = _rank(s2d, k)
    rankc = rankc2d.reshape(n32)
    scores = s_col.reshape(n32)
    perm_p, vals_p, hg = _sc_topk(rankc, scores, h, k, kp)
    perm = _reduce32(perm_p, "sum").reshape(kp)
    vals = _reduce32(vals_p, "sum").reshape(kp)
    return rankc, perm, vals, hg[0], hg[1]


def kernel(x, edge_index, edge_attr, params):
    src, dst = edge_index[0], edge_index[1]
    src_r = src.reshape(32, 125, 80)
    dst_r = dst.reshape(32, 125, 80)

    deg_parts = _sc_deg_parts(dst.reshape(32, E // 32))
    dinv = _reduce32(deg_parts, "dinv").reshape(N32, 1)

    x_pad = jnp.zeros((N32, x.shape[1]), jnp.float32).at[:N, :].set(x)

    def gcn_sparse(ins, W, b, relu, pre=None):
        Wp = W
        bp = b
        if W.shape[1] < H:
            Wp = jnp.zeros((W.shape[0], H), jnp.float32).at[:, :W.shape[1]].set(W)
            bp = jnp.zeros((H,), jnp.float32).at[:W.shape[1]].set(b)
        xs = _feat(ins, Wp, pre, dinv)
        p0, p1 = _sc_scatter_conv(xs, src_r, dst_r)
        return _sparse_combine(xs, p0, p1, dinv, bp.reshape(1, H), relu)

    h = gcn_sparse([x_pad], params['init_W'], params['init_b'], relu=True)
    h = gcn_sparse([h], params['down_W'][0], params['down_b'][0], relu=True)
    res0 = h

    k1, kp1 = 5000, 5120
    rank1, perm1, vals1, hg0, hg1 = _topk_level(h, params['pool_w'][0], N, k1, kp1)
    rks, rkd = _sc_edge_ranks(src.reshape(32, E // 32), dst.reshape(32, E // 32), rank1)
    srt_s, off_s = _sc_bucket(rks)
    srt_d, off_d = _sc_bucket(rkd)
    Rm = _sc_build_RC_sorted(srt_s, off_s, perm1, k1, kp1).reshape(kp1, N32)
    Ct = _sc_build_RC_sorted(srt_d, off_d, perm1, k1, kp1).reshape(kp1, N32)
    B1, Bt1 = _mmbig(Rm, Ct, k1, exact_bf16=True)
    dinv1 = _rowsum_dinv(Bt1)
    z = _feat([hg0, hg1], params['down_W'][1], vals1.reshape(kp1, 1), dinv1)
    h = _dense_combine(Bt1, z, dinv1, params['down_b'][1].reshape(1, H), relu=True)
    res1 = h

    k2, kp2 = 2500, 2560
    rank2, perm2, vals2, hg0, hg1 = _topk_level(h, params['pool_w'][1], k1, k2, kp2)
    R2, Ct2 = _sc_row_gather2(B1, Bt1, perm2)
    B2, Bt2 = _mmbig(R2, Ct2, k2)
    dinv2 = _rowsum_dinv(Bt2)
    z = _feat([hg0, hg1], params['down_W'][2], vals2.reshape(kp2, 1), dinv2)
    h = _dense_combine(Bt2, z, dinv2, params['down_b'][2].reshape(1, H), relu=True)
    res2 = h

    k3, kp3 = 1250, 1280
    rank3, perm3, vals3, hg0, hg1 = _topk_level(h, params['pool_w'][2], k2, k3, kp3)
    R3, Ct3 = _sc_row_gather2(B2, Bt2, perm3)
    B3, Bt3 = _mmbig(R3, Ct3, k3)
    dinv3 = _rowsum_dinv(Bt3)
    z = _feat([hg0, hg1], params['down_W'][3], vals3.reshape(kp3, 1), dinv3)
    h = _dense_combine(Bt3, z, dinv3, params['down_b'][3].reshape(1, H), relu=True)

    p0, p1 = _sc_up_scatter(perm3, h, k3, kp2, k2)
    z = _feat([res2, p0, p1], params['up_W'][0], None, dinv2)
    h = _dense_combine(Bt2, z, dinv2, params['up_b'][0].reshape(1, H), relu=True)

    p0, p1 = _sc_up_scatter(perm2, h, k2, kp1, k1)
    z = _feat([res1, p0, p1], params['up_W'][1], None, dinv1)
    h = _dense_combine(Bt1, z, dinv1, params['up_b'][1].reshape(1, H), relu=True)

    p0, p1 = _sc_up_scatter(perm1, h, k1, N32, N)
    h = gcn_sparse([res0, p0, p1], params['up_W'][2], params['up_b'][2], relu=False)

    h = gcn_sparse([h], params['W1'], params['b1'], relu=True)
    h = gcn_sparse([h], params['W2'], params['b2'], relu=False)

    return h[:N, :3].reshape(1, -1)
